# baseline (speedup 1.0000x reference)
import functools

import jax
import jax.numpy as jnp
from jax import lax
from jax.experimental import pallas as pl
from jax.experimental.pallas import tpu as pltpu
from jax.experimental.pallas import tpu_sc as plsc

NC = 2
NS = 16
NW = NC * NS
LANES = 16

CHUNK_BAGS = 4


def _make_kernel(B, H, D):
    bags_per_w = B // NW
    idx_per_chunk = CHUNK_BAGS * H
    nchunk = bags_per_w // CHUNK_BAGS
    col_groups = D // LANES
    inv_h = 1.0 / H

    mesh = plsc.VectorSubcoreMesh(core_axis_name="c", subcore_axis_name="s")

    nbuf = 4
    assert nchunk % nbuf == 0

    @functools.partial(
        pl.kernel,
        out_type=jax.ShapeDtypeStruct((B, D), jnp.float32),
        mesh=mesh,
        scratch_types=[
            pltpu.VMEM((nchunk, idx_per_chunk), jnp.int32),
            pltpu.VMEM((nbuf, idx_per_chunk, D), jnp.float32),
            pltpu.VMEM((bags_per_w, D), jnp.float32),
            [pltpu.SemaphoreType.DMA] * nbuf,
        ],
        compiler_params=pltpu.CompilerParams(use_tc_tiling_on_sc=False),
    )
    def bag_kernel(text_hbm, weight_hbm, out_hbm, idx_v, rows_v, out_v, sems):
        wid = lax.axis_index("s") * NC + lax.axis_index("c")
        pltpu.sync_copy(text_hbm.at[wid], idx_v)

        def start(j, b):
            pltpu.async_copy(weight_hbm.at[idx_v.at[j]], rows_v.at[b], sems[b])

        for b in range(nbuf):
            start(b, b)

        def outer(g, _):
            j0 = g * nbuf
            for b in range(nbuf):
                j = j0 + b
                pltpu.make_async_copy(
                    weight_hbm.at[idx_v.at[j]], rows_v.at[b], sems[b]
                ).wait()
                for bag in range(CHUNK_BAGS):
                    for c in range(col_groups):
                        acc = rows_v[b, bag * H, pl.ds(c * LANES, LANES)]
                        for r in range(1, H):
                            acc = acc + rows_v[b, bag * H + r, pl.ds(c * LANES, LANES)]
                        out_v[j * CHUNK_BAGS + bag, pl.ds(c * LANES, LANES)] = acc * inv_h
                nxt = j + nbuf

                @pl.when(nxt < nchunk)
                def _():
                    start(nxt, b)

            return 0

        lax.fori_loop(0, nchunk // nbuf, outer, 0)
        pltpu.sync_copy(out_v, out_hbm.at[pl.ds(wid * bags_per_w, bags_per_w)])

    return bag_kernel


def kernel(text, weight):
    B, H = text.shape
    _, D = weight.shape
    text_r = text.astype(jnp.int32).reshape(NW, (B // NW) // CHUNK_BAGS, CHUNK_BAGS * H)
    return _make_kernel(B, H, D)(text_r, weight)


# speedup vs baseline: 1.1768x; 1.1768x over previous
import functools

import jax
import jax.numpy as jnp
from jax import lax
from jax.experimental import pallas as pl
from jax.experimental.pallas import tpu as pltpu
from jax.experimental.pallas import tpu_sc as plsc

NC = 2
NS = 16
NW = NC * NS
LANES = 16

CHUNK_BAGS = 2


def _make_kernel(B, H, D):
    bags_per_w = B // NW
    idx_per_chunk = CHUNK_BAGS * H
    nchunk = bags_per_w // CHUNK_BAGS
    col_groups = D // LANES
    inv_h = 1.0 / H
    D2 = 2 * D
    off_groups = -(-idx_per_chunk // LANES)
    off_pad = off_groups * LANES

    mesh = plsc.VectorSubcoreMesh(core_axis_name="c", subcore_axis_name="s")

    nbuf = 2

    @functools.partial(
        pl.kernel,
        out_type=jax.ShapeDtypeStruct((B, D), jnp.float32),
        mesh=mesh,
        scratch_types=[
            pltpu.VMEM((nchunk, idx_per_chunk), jnp.int32),
            pltpu.VMEM((nchunk, off_pad), jnp.int32),
            pltpu.VMEM((nbuf, idx_per_chunk, D2), jnp.float32),
            pltpu.VMEM((bags_per_w, D), jnp.float32),
            [pltpu.SemaphoreType.DMA] * nbuf,
        ],
        compiler_params=pltpu.CompilerParams(use_tc_tiling_on_sc=False),
    )
    def bag_kernel(fr_hbm, off_hbm, wl_hbm, out_hbm, fr_v, off_v, rows_v, out_v, sems):
        wid = lax.axis_index("s") * NC + lax.axis_index("c")
        pltpu.sync_copy(fr_hbm.at[wid], fr_v)
        pltpu.sync_copy(off_hbm.at[wid], off_v)

        def start(j, b):
            pltpu.async_copy(wl_hbm.at[fr_v.at[j]], rows_v.at[b], sems[b])

        for b in range(nbuf):
            start(b, b)

        def chunk_body(j, b):
            pltpu.make_async_copy(
                wl_hbm.at[fr_v.at[j]], rows_v.at[b], sems[b]
            ).wait()
            offs = [off_v[j, pl.ds(k * LANES, LANES)] for k in range(off_groups)]
            for bag in range(CHUNK_BAGS):
                accs = None
                r0 = bag * H
                for r in range(H):
                    o = offs[(r0 + r) // LANES][(r0 + r) % LANES]
                    vals = [
                        rows_v[b, r0 + r, pl.ds(o + c * LANES, LANES)]
                        for c in range(col_groups)
                    ]
                    accs = vals if accs is None else [a + v for a, v in zip(accs, vals)]
                for c in range(col_groups):
                    out_v[j * CHUNK_BAGS + bag, pl.ds(c * LANES, LANES)] = accs[c] * inv_h

        def outer(g, _):
            j0 = g * nbuf
            for b in range(nbuf):
                j = j0 + b
                chunk_body(j, b)
                nxt = j + nbuf

                @pl.when(nxt < nchunk)
                def _():
                    start(nxt, b)

            return 0

        lax.fori_loop(0, nchunk // nbuf, outer, 0)
        for t in range(nchunk - nchunk % nbuf, nchunk):
            chunk_body(t, t % nbuf)
        pltpu.sync_copy(out_v, out_hbm.at[pl.ds(wid * bags_per_w, bags_per_w)])

    return bag_kernel


def kernel(text, weight):
    B, H = text.shape
    _, D = weight.shape
    t32 = text.astype(jnp.int32)
    nchunk = (B // NW) // CHUNK_BAGS
    ipc = CHUNK_BAGS * H
    pad = -(-ipc // LANES) * LANES - ipc
    fr = (t32 >> 1).reshape(NW, nchunk, ipc)
    off = ((t32 & 1) << 6).reshape(NW * nchunk, ipc)
    off = jnp.pad(off, ((0, 0), (0, pad))).reshape(NW, nchunk, ipc + pad)
    wl = weight.reshape(weight.shape[0] // 2, 2 * D)
    return _make_kernel(B, H, D)(fr, off, wl)
